# Initial kernel scaffold; baseline (speedup 1.0000x reference)
import jax
import jax.numpy as jnp
from jax.experimental import pallas as pl


def kernel(x, edge_index, batch, W1, b1, W2, b2, W3, b3, Wo, bo):
    raise NotImplementedError("write your pallas kernel here")


# speedup vs baseline: 1.0002x; 1.0002x over previous
import jax
import jax.numpy as jnp
from jax.experimental import pallas as pl

N = 100000
G = 64


def _softmax_kernel(p_ref, o_ref):
    z = p_ref[...]
    m = jnp.max(z, axis=1, keepdims=True)
    e = jnp.exp(z - m)
    o_ref[...] = e / jnp.sum(e, axis=1, keepdims=True)


def kernel(x, edge_index, batch, W1, b1, W2, b2, W3, b3, Wo, bo):
    loop = jnp.arange(N, dtype=edge_index.dtype)
    src = jnp.concatenate([edge_index[0], loop])
    dst = jnp.concatenate([edge_index[1], loop])
    ones = jnp.ones(src.shape[0], dtype=jnp.float32)
    deg = jnp.zeros((N,), dtype=jnp.float32).at[dst].add(ones)
    dinv = jnp.where(deg > 0, jax.lax.rsqrt(jnp.maximum(deg, 1.0)), 0.0)
    norm = dinv[src] * dinv[dst]
    h = x
    for W, b in ((W1, b1), (W2, b2), (W3, b3)):
        hw = h @ W
        msg = hw[src] * norm[:, None]
        h = jnp.zeros((N, W.shape[1]), dtype=hw.dtype).at[dst].add(msg) + b
        h = jax.nn.relu(h)
    h = h @ Wo + bo
    pooled = jax.ops.segment_max(h, batch, num_segments=G)
    return pl.pallas_call(
        _softmax_kernel,
        out_shape=jax.ShapeDtypeStruct((G, 3), jnp.float32),
    )(pooled)


# speedup vs baseline: 13.3577x; 13.3555x over previous
import functools

import jax
import jax.numpy as jnp
from jax import lax
from jax.experimental import pallas as pl
from jax.experimental.pallas import tpu as pltpu
from jax.experimental.pallas import tpu_sc as plsc

N = 100000
E = 1600000
NGROUPS = 64

NW = 32
EPT = E // NW
WINDOW = 2000
NWIN = EPT // WINDOW
NCH = 5
CH = WINDOW // NCH
SUB = 6248
TAIL = N - 16 * SUB
ZROWS = 71

BN = 4000
NSTEP = N // BN

_mesh = plsc.VectorSubcoreMesh(core_axis_name="c", subcore_axis_name="s")
_sc_params = pltpu.CompilerParams(use_tc_tiling_on_sc=False)


def _make_sc_agg(G):
    @functools.partial(
        pl.kernel,
        out_type=jax.ShapeDtypeStruct((2, G, N, 16), jnp.float32),
        mesh=_mesh,
        compiler_params=_sc_params,
        scratch_types=[
            pltpu.VMEM_SHARED((N, 16), jnp.float32),
            pltpu.VMEM((NCH, CH), jnp.int32),
            pltpu.VMEM((NCH, CH), jnp.int32),
            pltpu.VMEM((CH,), jnp.int32),
            pltpu.VMEM((CH, 16), jnp.float32),
            pltpu.VMEM((ZROWS, 16), jnp.float32),
            pltpu.SemaphoreType.DMA,
        ],
    )
    def agg(xs_hbm, src_hbm, dst_hbm, out_hbm,
            accum, src_v, dst_v, idx_v, rows_v, z_v, sem):
        c = lax.axis_index("c")
        s = lax.axis_index("s")
        t = c * 16 + s

        def zb(i, carry):
            z_v[i] = jnp.zeros((16,), jnp.float32)
            return carry

        lax.fori_loop(0, ZROWS, zb, 0)

        for g in range(G):
            for k in range(SUB // ZROWS):
                pltpu.sync_copy(
                    z_v, accum.at[pl.ds(s * SUB + k * ZROWS, ZROWS)])

            @pl.when(s == 15)
            def _():
                pltpu.sync_copy(z_v.at[pl.ds(0, TAIL)],
                                accum.at[pl.ds(16 * SUB, TAIL)])

            plsc.subcore_barrier()

            def win(w, carry):
                pltpu.sync_copy(src_hbm.at[t, w], src_v)
                pltpu.sync_copy(dst_hbm.at[t, w], dst_v)
                for k in range(NCH):
                    for j in range(CH // 16):
                        idx_v[pl.ds(j * 16, 16)] = (
                            src_v[k, pl.ds(j * 16, 16)] * G + g)
                    pltpu.async_copy(xs_hbm.at[idx_v], rows_v, sem).wait()
                    pltpu.sync_copy(
                        rows_v, accum.at[dst_v.at[k]], add=True)
                return carry

            lax.fori_loop(0, NWIN, win, 0)
            plsc.subcore_barrier()
            pltpu.sync_copy(
                accum.at[pl.ds(s * SUB, SUB)],
                out_hbm.at[c, g, pl.ds(s * SUB, SUB)])

            @pl.when(s == 15)
            def _():
                pltpu.sync_copy(accum.at[pl.ds(16 * SUB, TAIL)],
                                out_hbm.at[c, g, pl.ds(16 * SUB, TAIL)])

            plsc.subcore_barrier()

    return agg


@functools.partial(
    pl.kernel,
    out_type=jax.ShapeDtypeStruct((2, N, 16), jnp.float32),
    mesh=_mesh,
    compiler_params=_sc_params,
    scratch_types=[
        pltpu.VMEM_SHARED((N, 16), jnp.float32),
        pltpu.VMEM((NCH, CH), jnp.int32),
        pltpu.VMEM((CH, 16), jnp.float32),
        pltpu.VMEM((ZROWS, 16), jnp.float32),
    ],
)
def _sc_degree(dst_hbm, out_hbm, accum, dst_v, ones_v, z_v):
    c = lax.axis_index("c")
    s = lax.axis_index("s")
    t = c * 16 + s

    def zb(i, carry):
        z_v[i] = jnp.zeros((16,), jnp.float32)
        return carry

    lax.fori_loop(0, ZROWS, zb, 0)

    def ob(i, carry):
        ones_v[i] = jnp.ones((16,), jnp.float32)
        return carry

    lax.fori_loop(0, CH, ob, 0)

    for k in range(SUB // ZROWS):
        pltpu.sync_copy(
            z_v, accum.at[pl.ds(s * SUB + k * ZROWS, ZROWS)])

    @pl.when(s == 15)
    def _():
        pltpu.sync_copy(z_v.at[pl.ds(0, TAIL)],
                        accum.at[pl.ds(16 * SUB, TAIL)])

    plsc.subcore_barrier()

    def win(w, carry):
        pltpu.sync_copy(dst_hbm.at[t, w], dst_v)
        for k in range(NCH):
            pltpu.sync_copy(ones_v, accum.at[dst_v.at[k]], add=True)
        return carry

    lax.fori_loop(0, NWIN, win, 0)
    plsc.subcore_barrier()
    pltpu.sync_copy(
        accum.at[pl.ds(s * SUB, SUB)],
        out_hbm.at[c, pl.ds(s * SUB, SUB)])

    @pl.when(s == 15)
    def _():
        pltpu.sync_copy(accum.at[pl.ds(16 * SUB, TAIL)],
                        out_hbm.at[c, pl.ds(16 * SUB, TAIL)])


_sc_agg3 = _make_sc_agg(3)
_sc_agg5 = _make_sc_agg(5)
_sc_agg4 = _make_sc_agg(4)


def _tc0_body(deg_ref, x_ref, dinv_ref, xs1_ref):
    i = pl.program_id(0)
    d = deg_ref[...]
    deg = d[0, :, 0] + d[1, :, 0] + 1.0
    dv = lax.rsqrt(deg)
    dinv_ref[...] = dv.reshape(1, 1, BN)
    xs = x_ref[...] * dv[:, None]
    xs1_ref[...] = jnp.concatenate(
        [xs, jnp.zeros((BN, 12), jnp.float32)], axis=1)


def _tc_layer_body(G, dout, pad_out, two_mm,
                   agg_ref, xs_ref, dinv_ref, w_ref, b_ref, w2_ref, out_ref):
    dv = dinv_ref[...].reshape(BN)
    a = agg_ref[...]
    sg = a[0] + a[1]
    u = jnp.concatenate([sg[g] for g in range(G)], axis=1)
    pre = (u + xs_ref[...]) * dv[:, None]
    h = jnp.dot(pre, w_ref[...], preferred_element_type=jnp.float32)
    h = jnp.maximum(h + b_ref[...][None, :], 0.0)
    if two_mm:
        h = jnp.dot(h, w2_ref[...], preferred_element_type=jnp.float32)
    hs = h * dv[:, None]
    if pad_out:
        hs = jnp.concatenate(
            [hs, jnp.zeros((BN, pad_out), jnp.float32)], axis=1)
    out_ref[...] = hs


def _tc3_body(agg_ref, xs_ref, dinv_ref, b3_ref, wot_ref, bo_ref, batch_ref,
              out_ref, pool_ref):
    i = pl.program_id(0)
    dv = dinv_ref[...].reshape(BN)
    a = agg_ref[...]
    sg = a[0] + a[1]
    u = jnp.concatenate([sg[g] for g in range(4)], axis=1)
    a3 = (u + xs_ref[...]) * dv[:, None]
    h3 = jnp.maximum(a3 + b3_ref[...][None, :], 0.0)
    zt = lax.dot_general(
        wot_ref[...], h3, (((1,), (1,)), ((), ())),
        preferred_element_type=jnp.float32) + bo_ref[...][:, None]
    bb = batch_ref[...].reshape(BN)
    m = bb[None, :] == lax.broadcasted_iota(jnp.int32, (NGROUPS, BN), 0)
    neg = jnp.float32(-jnp.inf)

    @pl.when(i == 0)
    def _():
        pool_ref[...] = jnp.full((3, NGROUPS), neg, jnp.float32)

    for col in range(3):
        colmax = jnp.max(jnp.where(m, zt[col][None, :], neg), axis=1)
        pool_ref[col, :] = jnp.maximum(pool_ref[col, :], colmax)

    @pl.when(i == NSTEP - 1)
    def _():
        p = pool_ref[...]
        mx = jnp.max(p, axis=0, keepdims=True)
        ex = jnp.exp(p - mx)
        out_ref[...] = ex / jnp.sum(ex, axis=0, keepdims=True)


_arb = pltpu.CompilerParams(dimension_semantics=("arbitrary",))


def _tc0(deg2, x):
    return pl.pallas_call(
        _tc0_body,
        grid=(NSTEP,),
        in_specs=[
            pl.BlockSpec((2, BN, 16), lambda i: (0, i, 0)),
            pl.BlockSpec((BN, 36), lambda i: (i, 0)),
        ],
        out_specs=[
            pl.BlockSpec((1, 1, BN), lambda i: (i, 0, 0)),
            pl.BlockSpec((BN, 48), lambda i: (i, 0)),
        ],
        out_shape=[
            jax.ShapeDtypeStruct((NSTEP, 1, BN), jnp.float32),
            jax.ShapeDtypeStruct((N, 48), jnp.float32),
        ],
        compiler_params=_arb,
    )(deg2, x)


def _tc_layer(agg, xs, dinv, w, b, w2, G, din_pad, dout, pad_out):
    two_mm = w2 is not None
    if w2 is None:
        w2 = jnp.zeros((1, 1), jnp.float32)
    body = functools.partial(_tc_layer_body, G, dout, pad_out, two_mm)
    return pl.pallas_call(
        body,
        grid=(NSTEP,),
        in_specs=[
            pl.BlockSpec((2, G, BN, 16), lambda i: (0, 0, i, 0)),
            pl.BlockSpec((BN, din_pad), lambda i: (i, 0)),
            pl.BlockSpec((1, 1, BN), lambda i: (i, 0, 0)),
            pl.BlockSpec(w.shape, lambda i: (0, 0)),
            pl.BlockSpec(b.shape, lambda i: (0,)),
            pl.BlockSpec(w2.shape, lambda i: (0, 0)),
        ],
        out_specs=pl.BlockSpec((BN, dout + pad_out), lambda i: (i, 0)),
        out_shape=jax.ShapeDtypeStruct((N, dout + pad_out), jnp.float32),
        compiler_params=_arb,
    )(agg, xs, dinv, w, b, w2)


def _tc3(agg, xs, dinv, b3p, wot, bo, batch):
    return pl.pallas_call(
        _tc3_body,
        grid=(NSTEP,),
        in_specs=[
            pl.BlockSpec((2, 4, BN, 16), lambda i: (0, 0, i, 0)),
            pl.BlockSpec((BN, 64), lambda i: (i, 0)),
            pl.BlockSpec((1, 1, BN), lambda i: (i, 0, 0)),
            pl.BlockSpec((64,), lambda i: (0,)),
            pl.BlockSpec((3, 64), lambda i: (0, 0)),
            pl.BlockSpec((3,), lambda i: (0,)),
            pl.BlockSpec((1, 1, BN), lambda i: (i, 0, 0)),
        ],
        out_specs=pl.BlockSpec((3, NGROUPS), lambda i: (0, 0)),
        out_shape=jax.ShapeDtypeStruct((3, NGROUPS), jnp.float32),
        scratch_shapes=[pltpu.VMEM((3, NGROUPS), jnp.float32)],
        compiler_params=_arb,
    )(agg, xs, dinv, b3p, wot, bo, batch)


def kernel(x, edge_index, batch, W1, b1, W2, b2, W3, b3, Wo, bo):
    src3 = edge_index[0].reshape(NW, NWIN, NCH, CH)
    dst3 = edge_index[1].reshape(NW, NWIN, NCH, CH)

    W1p = jnp.pad(W1, ((0, 12), (0, 0)))
    W2p = jnp.pad(W2, ((0, 5), (0, 0)))
    W3p = jnp.pad(W3, ((0, 0), (0, 14)))
    b3p = jnp.pad(b3, (0, 14))
    WoT = jnp.pad(Wo, ((0, 14), (0, 0))).T

    deg2 = _sc_degree(dst3)
    dinv, xs1 = _tc0(deg2, x)

    agg1 = _sc_agg3(xs1.reshape(N * 3, 16), src3, dst3)
    xs2 = _tc_layer(agg1, xs1, dinv, W1p, b1, None, 3, 48, 75, 5)

    agg2 = _sc_agg5(xs2.reshape(N * 5, 16), src3, dst3)
    xs3 = _tc_layer(agg2, xs2, dinv, W2p, b2, W3p, 5, 80, 64, 0)

    agg3 = _sc_agg4(xs3.reshape(N * 4, 16), src3, dst3)
    pooled_t = _tc3(agg3, xs3, dinv, b3p, WoT, bo,
                    batch.reshape(NSTEP, 1, BN))
    return pooled_t.T


# speedup vs baseline: 15.0888x; 1.1296x over previous
import functools

import jax
import jax.numpy as jnp
from jax import lax
from jax.experimental import pallas as pl
from jax.experimental.pallas import tpu as pltpu
from jax.experimental.pallas import tpu_sc as plsc

N = 100000
E = 1600000
NGROUPS = 64

NW = 32
EPT = E // NW
CH = 400
NWCH = EPT // CH
NWIN = 25
NCH = 5
SUB = 6248
TAIL = N - 16 * SUB
ZROWS = 71

BN = 4000
NSTEP = N // BN

_mesh = plsc.VectorSubcoreMesh(core_axis_name="c", subcore_axis_name="s")
_sc_params = pltpu.CompilerParams(use_tc_tiling_on_sc=False)


def _make_sc_agg(G):
    @functools.partial(
        pl.kernel,
        out_type=jax.ShapeDtypeStruct((2, G, N, 16), jnp.float32),
        mesh=_mesh,
        compiler_params=_sc_params,
        scratch_types=[
            pltpu.VMEM_SHARED((N, 16), jnp.float32),
            pltpu.VMEM((2, 2, 1, CH), jnp.int32),
            pltpu.VMEM((2, CH), jnp.int32),
            pltpu.VMEM((2, CH), jnp.int32),
            pltpu.VMEM((2, CH, 16), jnp.float32),
            pltpu.VMEM((ZROWS, 16), jnp.float32),
            pltpu.SemaphoreType.DMA((2,)),
            pltpu.SemaphoreType.DMA((2,)),
            pltpu.SemaphoreType.DMA((2,)),
        ],
    )
    def agg(xs_hbm, ed_hbm, out_hbm,
            accum, ed_v, idx_v, didx_v, rows_v, z_v, esem, gsem, ssem):
        c = lax.axis_index("c")
        s = lax.axis_index("s")
        t = c * 16 + s

        def zb(i, carry):
            z_v[i] = jnp.zeros((16,), jnp.float32)
            return carry

        lax.fori_loop(0, ZROWS, zb, 0)

        def e_cp(k, b):
            return pltpu.make_async_copy(
                ed_hbm.at[:, t, k], ed_v.at[b], esem.at[b])

        def g_cp(b):
            return pltpu.make_async_copy(
                xs_hbm.at[idx_v.at[b]], rows_v.at[b], gsem.at[b])

        def s_cp(b):
            return pltpu.make_async_copy(
                rows_v.at[b], accum.at[didx_v.at[b]], ssem.at[b])

        for g in range(G):

            def idx_compute(b):
                for j in range(CH // 16):
                    sl = pl.ds(j * 16, 16)
                    idx_v[b, sl] = ed_v[b, 0, 0, sl] * G + g
                    didx_v[b, sl] = ed_v[b, 1, 0, sl]

            for k in range(SUB // ZROWS):
                pltpu.sync_copy(
                    z_v, accum.at[pl.ds(s * SUB + k * ZROWS, ZROWS)])

            @pl.when(s == 15)
            def _():
                pltpu.sync_copy(z_v.at[pl.ds(0, TAIL)],
                                accum.at[pl.ds(16 * SUB, TAIL)])

            plsc.subcore_barrier()

            e_cp(0, 0).start()
            e_cp(0, 0).wait()
            idx_compute(0)
            pltpu.async_copy(
                xs_hbm.at[idx_v.at[0]], rows_v.at[0], gsem.at[0])
            e_cp(1, 1).start()

            def step(k, carry):
                b = jnp.bitwise_and(k, 1)
                nb = 1 - b
                g_cp(b).wait()
                pltpu.async_copy(
                    rows_v.at[b], accum.at[didx_v.at[b]], ssem.at[b],
                    add=True)

                @pl.when(k + 1 < NWCH)
                def _():
                    @pl.when(k > 0)
                    def _():
                        s_cp(nb).wait()

                    e_cp(k + 1, nb).wait()
                    idx_compute(nb)

                    @pl.when(k + 2 < NWCH)
                    def _():
                        e_cp(k + 2, b).start()

                    pltpu.async_copy(
                        xs_hbm.at[idx_v.at[nb]], rows_v.at[nb], gsem.at[nb])

                return carry

            lax.fori_loop(0, NWCH, step, 0)
            s_cp(jnp.int32((NWCH - 2) & 1)).wait()
            s_cp(jnp.int32((NWCH - 1) & 1)).wait()
            plsc.subcore_barrier()
            pltpu.sync_copy(
                accum.at[pl.ds(s * SUB, SUB)],
                out_hbm.at[c, g, pl.ds(s * SUB, SUB)])

            @pl.when(s == 15)
            def _():
                pltpu.sync_copy(accum.at[pl.ds(16 * SUB, TAIL)],
                                out_hbm.at[c, g, pl.ds(16 * SUB, TAIL)])

            plsc.subcore_barrier()

    return agg


@functools.partial(
    pl.kernel,
    out_type=jax.ShapeDtypeStruct((2, N, 16), jnp.float32),
    mesh=_mesh,
    compiler_params=_sc_params,
    scratch_types=[
        pltpu.VMEM_SHARED((N, 16), jnp.float32),
        pltpu.VMEM((NCH, CH), jnp.int32),
        pltpu.VMEM((CH, 16), jnp.float32),
        pltpu.VMEM((ZROWS, 16), jnp.float32),
    ],
)
def _sc_degree(dst_hbm, out_hbm, accum, dst_v, ones_v, z_v):
    c = lax.axis_index("c")
    s = lax.axis_index("s")
    t = c * 16 + s

    def zb(i, carry):
        z_v[i] = jnp.zeros((16,), jnp.float32)
        return carry

    lax.fori_loop(0, ZROWS, zb, 0)

    def ob(i, carry):
        ones_v[i] = jnp.ones((16,), jnp.float32)
        return carry

    lax.fori_loop(0, CH, ob, 0)

    for k in range(SUB // ZROWS):
        pltpu.sync_copy(
            z_v, accum.at[pl.ds(s * SUB + k * ZROWS, ZROWS)])

    @pl.when(s == 15)
    def _():
        pltpu.sync_copy(z_v.at[pl.ds(0, TAIL)],
                        accum.at[pl.ds(16 * SUB, TAIL)])

    plsc.subcore_barrier()

    def win(w, carry):
        pltpu.sync_copy(dst_hbm.at[t, w], dst_v)
        for k in range(NCH):
            pltpu.sync_copy(ones_v, accum.at[dst_v.at[k]], add=True)
        return carry

    lax.fori_loop(0, NWIN, win, 0)
    plsc.subcore_barrier()
    pltpu.sync_copy(
        accum.at[pl.ds(s * SUB, SUB)],
        out_hbm.at[c, pl.ds(s * SUB, SUB)])

    @pl.when(s == 15)
    def _():
        pltpu.sync_copy(accum.at[pl.ds(16 * SUB, TAIL)],
                        out_hbm.at[c, pl.ds(16 * SUB, TAIL)])


_sc_agg3 = _make_sc_agg(3)
_sc_agg5 = _make_sc_agg(5)
_sc_agg4 = _make_sc_agg(4)


def _tc0_body(deg_ref, x_ref, dinv_ref, xs1_ref):
    i = pl.program_id(0)
    d = deg_ref[...]
    deg = d[0, :, 0] + d[1, :, 0] + 1.0
    dv = lax.rsqrt(deg)
    dinv_ref[...] = dv.reshape(1, 1, BN)
    xs = x_ref[...] * dv[:, None]
    xs1_ref[...] = jnp.concatenate(
        [xs, jnp.zeros((BN, 12), jnp.float32)], axis=1)


def _tc_layer_body(G, dout, pad_out, two_mm,
                   agg_ref, xs_ref, dinv_ref, w_ref, b_ref, w2_ref, out_ref):
    dv = dinv_ref[...].reshape(BN)
    a = agg_ref[...]
    sg = a[0] + a[1]
    u = jnp.concatenate([sg[g] for g in range(G)], axis=1)
    pre = (u + xs_ref[...]) * dv[:, None]
    h = jnp.dot(pre, w_ref[...], preferred_element_type=jnp.float32)
    h = jnp.maximum(h + b_ref[...][None, :], 0.0)
    if two_mm:
        h = jnp.dot(h, w2_ref[...], preferred_element_type=jnp.float32)
    hs = h * dv[:, None]
    if pad_out:
        hs = jnp.concatenate(
            [hs, jnp.zeros((BN, pad_out), jnp.float32)], axis=1)
    out_ref[...] = hs


def _tc3_body(agg_ref, xs_ref, dinv_ref, b3_ref, wot_ref, bo_ref, batch_ref,
              out_ref, pool_ref):
    i = pl.program_id(0)
    dv = dinv_ref[...].reshape(BN)
    a = agg_ref[...]
    sg = a[0] + a[1]
    u = jnp.concatenate([sg[g] for g in range(4)], axis=1)
    a3 = (u + xs_ref[...]) * dv[:, None]
    h3 = jnp.maximum(a3 + b3_ref[...][None, :], 0.0)
    zt = lax.dot_general(
        wot_ref[...], h3, (((1,), (1,)), ((), ())),
        preferred_element_type=jnp.float32) + bo_ref[...][:, None]
    bb = batch_ref[...].reshape(BN)
    m = bb[None, :] == lax.broadcasted_iota(jnp.int32, (NGROUPS, BN), 0)
    neg = jnp.float32(-jnp.inf)

    @pl.when(i == 0)
    def _():
        pool_ref[...] = jnp.full((3, NGROUPS), neg, jnp.float32)

    for col in range(3):
        colmax = jnp.max(jnp.where(m, zt[col][None, :], neg), axis=1)
        pool_ref[col, :] = jnp.maximum(pool_ref[col, :], colmax)

    @pl.when(i == NSTEP - 1)
    def _():
        p = pool_ref[...]
        mx = jnp.max(p, axis=0, keepdims=True)
        ex = jnp.exp(p - mx)
        out_ref[...] = ex / jnp.sum(ex, axis=0, keepdims=True)


_arb = pltpu.CompilerParams(dimension_semantics=("arbitrary",))


def _tc0(deg2, x):
    return pl.pallas_call(
        _tc0_body,
        grid=(NSTEP,),
        in_specs=[
            pl.BlockSpec((2, BN, 16), lambda i: (0, i, 0)),
            pl.BlockSpec((BN, 36), lambda i: (i, 0)),
        ],
        out_specs=[
            pl.BlockSpec((1, 1, BN), lambda i: (i, 0, 0)),
            pl.BlockSpec((BN, 48), lambda i: (i, 0)),
        ],
        out_shape=[
            jax.ShapeDtypeStruct((NSTEP, 1, BN), jnp.float32),
            jax.ShapeDtypeStruct((N, 48), jnp.float32),
        ],
        compiler_params=_arb,
    )(deg2, x)


def _tc_layer(agg, xs, dinv, w, b, w2, G, din_pad, dout, pad_out):
    two_mm = w2 is not None
    if w2 is None:
        w2 = jnp.zeros((1, 1), jnp.float32)
    body = functools.partial(_tc_layer_body, G, dout, pad_out, two_mm)
    return pl.pallas_call(
        body,
        grid=(NSTEP,),
        in_specs=[
            pl.BlockSpec((2, G, BN, 16), lambda i: (0, 0, i, 0)),
            pl.BlockSpec((BN, din_pad), lambda i: (i, 0)),
            pl.BlockSpec((1, 1, BN), lambda i: (i, 0, 0)),
            pl.BlockSpec(w.shape, lambda i: (0, 0)),
            pl.BlockSpec(b.shape, lambda i: (0,)),
            pl.BlockSpec(w2.shape, lambda i: (0, 0)),
        ],
        out_specs=pl.BlockSpec((BN, dout + pad_out), lambda i: (i, 0)),
        out_shape=jax.ShapeDtypeStruct((N, dout + pad_out), jnp.float32),
        compiler_params=_arb,
    )(agg, xs, dinv, w, b, w2)


def _tc3(agg, xs, dinv, b3p, wot, bo, batch):
    return pl.pallas_call(
        _tc3_body,
        grid=(NSTEP,),
        in_specs=[
            pl.BlockSpec((2, 4, BN, 16), lambda i: (0, 0, i, 0)),
            pl.BlockSpec((BN, 64), lambda i: (i, 0)),
            pl.BlockSpec((1, 1, BN), lambda i: (i, 0, 0)),
            pl.BlockSpec((64,), lambda i: (0,)),
            pl.BlockSpec((3, 64), lambda i: (0, 0)),
            pl.BlockSpec((3,), lambda i: (0,)),
            pl.BlockSpec((1, 1, BN), lambda i: (i, 0, 0)),
        ],
        out_specs=pl.BlockSpec((3, NGROUPS), lambda i: (0, 0)),
        out_shape=jax.ShapeDtypeStruct((3, NGROUPS), jnp.float32),
        scratch_shapes=[pltpu.VMEM((3, NGROUPS), jnp.float32)],
        compiler_params=_arb,
    )(agg, xs, dinv, b3p, wot, bo, batch)


def kernel(x, edge_index, batch, W1, b1, W2, b2, W3, b3, Wo, bo):
    ed5 = edge_index.reshape(2, NW, NWCH, 1, CH)
    dst3 = edge_index[1].reshape(NW, NWIN, NCH, CH)

    W1p = jnp.pad(W1, ((0, 12), (0, 0)))
    W2p = jnp.pad(W2, ((0, 5), (0, 0)))
    W3p = jnp.pad(W3, ((0, 0), (0, 14)))
    b3p = jnp.pad(b3, (0, 14))
    WoT = jnp.pad(Wo, ((0, 14), (0, 0))).T

    deg2 = _sc_degree(dst3)
    dinv, xs1 = _tc0(deg2, x)

    agg1 = _sc_agg3(xs1.reshape(N * 3, 16), ed5)
    xs2 = _tc_layer(agg1, xs1, dinv, W1p, b1, None, 3, 48, 75, 5)

    agg2 = _sc_agg5(xs2.reshape(N * 5, 16), ed5)
    xs3 = _tc_layer(agg2, xs2, dinv, W2p, b2, W3p, 5, 80, 64, 0)

    agg3 = _sc_agg4(xs3.reshape(N * 4, 16), ed5)
    pooled_t = _tc3(agg3, xs3, dinv, b3p, WoT, bo,
                    batch.reshape(NSTEP, 1, BN))
    return pooled_t.T


# speedup vs baseline: 21.1861x; 1.4041x over previous
import functools

import jax
import jax.numpy as jnp
from jax import lax
from jax.experimental import pallas as pl
from jax.experimental import pallas as pl
from jax.experimental.pallas import tpu as pltpu
from jax.experimental.pallas import tpu_sc as plsc

N = 100000
NP = 100096
E = 1600000
NGROUPS = 64

NW = 32
EPT = E // NW
CH = 400
NWCH = EPT // CH
NWIN = 25
NCH = 5
SUB = 6272
SUBL = NP - 15 * SUB
ZR = 448

BN = 4352
NSTEP = NP // BN
RB = BN // 8
R = NP // 8

_mesh = plsc.VectorSubcoreMesh(core_axis_name="c", subcore_axis_name="s")
_sc_params = pltpu.CompilerParams(use_tc_tiling_on_sc=False)


def _make_sc_agg(G):
    @functools.partial(
        pl.kernel,
        out_type=jax.ShapeDtypeStruct((2, G, NP, 16), jnp.float32),
        mesh=_mesh,
        compiler_params=_sc_params,
        scratch_types=[
            pltpu.VMEM_SHARED((NP, 16), jnp.float32),
            pltpu.VMEM((2, 2, 1, CH), jnp.int32),
            pltpu.VMEM((2, CH), jnp.int32),
            pltpu.VMEM((2, CH), jnp.int32),
            pltpu.VMEM((2, CH, 16), jnp.float32),
            pltpu.VMEM((ZR, 16), jnp.float32),
            pltpu.SemaphoreType.DMA((2,)),
            pltpu.SemaphoreType.DMA((2,)),
            pltpu.SemaphoreType.DMA((2,)),
        ],
    )
    def agg(xs_hbm, ed_hbm, out_hbm,
            accum, ed_v, idx_v, didx_v, rows_v, z_v, esem, gsem, ssem):
        c = lax.axis_index("c")
        s = lax.axis_index("s")
        t = c * 16 + s

        def zb(i, carry):
            z_v[i] = jnp.zeros((16,), jnp.float32)
            return carry

        lax.fori_loop(0, ZR, zb, 0)

        def e_cp(k, b):
            return pltpu.make_async_copy(
                ed_hbm.at[:, t, k], ed_v.at[b], esem.at[b])

        def g_cp(b):
            return pltpu.make_async_copy(
                xs_hbm.at[idx_v.at[b]], rows_v.at[b], gsem.at[b])

        def s_cp(b):
            return pltpu.make_async_copy(
                rows_v.at[b], accum.at[didx_v.at[b]], ssem.at[b])

        for g in range(G):
            base = g * NP

            def idx_compute(b):
                for j in range(CH // 16):
                    sl = pl.ds(j * 16, 16)
                    idx_v[b, sl] = ed_v[b, 0, 0, sl] + base
                    didx_v[b, sl] = ed_v[b, 1, 0, sl]

            @pl.when(s < 15)
            def _():
                for k in range(SUB // ZR):
                    pltpu.sync_copy(
                        z_v, accum.at[pl.ds(s * SUB + k * ZR, ZR)])

            @pl.when(s == 15)
            def _():
                for k in range(SUBL // ZR):
                    pltpu.sync_copy(
                        z_v, accum.at[pl.ds(15 * SUB + k * ZR, ZR)])
                pltpu.sync_copy(
                    z_v.at[pl.ds(0, SUBL - ZR * (SUBL // ZR))],
                    accum.at[pl.ds(15 * SUB + ZR * (SUBL // ZR),
                                   SUBL - ZR * (SUBL // ZR))])

            plsc.subcore_barrier()

            e_cp(0, 0).start()
            e_cp(0, 0).wait()
            idx_compute(0)
            pltpu.async_copy(
                xs_hbm.at[idx_v.at[0]], rows_v.at[0], gsem.at[0])
            e_cp(1, 1).start()

            def step(k, carry):
                b = jnp.bitwise_and(k, 1)
                nb = 1 - b
                g_cp(b).wait()
                pltpu.async_copy(
                    rows_v.at[b], accum.at[didx_v.at[b]], ssem.at[b],
                    add=True)

                @pl.when(k + 1 < NWCH)
                def _():
                    @pl.when(k > 0)
                    def _():
                        s_cp(nb).wait()

                    e_cp(k + 1, nb).wait()
                    idx_compute(nb)

                    @pl.when(k + 2 < NWCH)
                    def _():
                        e_cp(k + 2, b).start()

                    pltpu.async_copy(
                        xs_hbm.at[idx_v.at[nb]], rows_v.at[nb], gsem.at[nb])

                return carry

            lax.fori_loop(0, NWCH, step, 0)
            s_cp(jnp.int32((NWCH - 2) & 1)).wait()
            s_cp(jnp.int32((NWCH - 1) & 1)).wait()
            plsc.subcore_barrier()

            @pl.when(s < 15)
            def _():
                pltpu.sync_copy(
                    accum.at[pl.ds(s * SUB, SUB)],
                    out_hbm.at[c, g, pl.ds(s * SUB, SUB)])

            @pl.when(s == 15)
            def _():
                pltpu.sync_copy(
                    accum.at[pl.ds(15 * SUB, SUBL)],
                    out_hbm.at[c, g, pl.ds(15 * SUB, SUBL)])

            plsc.subcore_barrier()

    return agg


@functools.partial(
    pl.kernel,
    out_type=jax.ShapeDtypeStruct((2, NP, 16), jnp.float32),
    mesh=_mesh,
    compiler_params=_sc_params,
    scratch_types=[
        pltpu.VMEM_SHARED((NP, 16), jnp.float32),
        pltpu.VMEM((NCH, CH), jnp.int32),
        pltpu.VMEM((CH, 16), jnp.float32),
        pltpu.VMEM((ZR, 16), jnp.float32),
    ],
)
def _sc_degree(dst_hbm, out_hbm, accum, dst_v, ones_v, z_v):
    c = lax.axis_index("c")
    s = lax.axis_index("s")
    t = c * 16 + s

    def zb(i, carry):
        z_v[i] = jnp.zeros((16,), jnp.float32)
        return carry

    lax.fori_loop(0, ZR, zb, 0)

    def ob(i, carry):
        ones_v[i] = jnp.ones((16,), jnp.float32)
        return carry

    lax.fori_loop(0, CH, ob, 0)

    @pl.when(s < 15)
    def _():
        for k in range(SUB // ZR):
            pltpu.sync_copy(z_v, accum.at[pl.ds(s * SUB + k * ZR, ZR)])

    @pl.when(s == 15)
    def _():
        for k in range(SUBL // ZR):
            pltpu.sync_copy(z_v, accum.at[pl.ds(15 * SUB + k * ZR, ZR)])
        pltpu.sync_copy(
            z_v.at[pl.ds(0, SUBL - ZR * (SUBL // ZR))],
            accum.at[pl.ds(15 * SUB + ZR * (SUBL // ZR),
                           SUBL - ZR * (SUBL // ZR))])

    plsc.subcore_barrier()

    def win(w, carry):
        pltpu.sync_copy(dst_hbm.at[t, w], dst_v)
        for k in range(NCH):
            pltpu.sync_copy(ones_v, accum.at[dst_v.at[k]], add=True)
        return carry

    lax.fori_loop(0, NWIN, win, 0)
    plsc.subcore_barrier()

    @pl.when(s < 15)
    def _():
        pltpu.sync_copy(
            accum.at[pl.ds(s * SUB, SUB)],
            out_hbm.at[c, pl.ds(s * SUB, SUB)])

    @pl.when(s == 15)
    def _():
        pltpu.sync_copy(
            accum.at[pl.ds(15 * SUB, SUBL)],
            out_hbm.at[c, pl.ds(15 * SUB, SUBL)])


_sc_agg3 = _make_sc_agg(3)
_sc_agg5 = _make_sc_agg(5)
_sc_agg4 = _make_sc_agg(4)


def _tc0_body(deg_ref, xpk_ref, dinv_ref, xs1_ref):
    d = deg_ref[...]
    dvp = lax.rsqrt(d[0] + d[1] + 1.0)
    dinv_ref[...] = dvp
    for g in range(3):
        xs1_ref[g] = xpk_ref[g] * dvp


def _tc_layer_body(gin, gmid, gout, two_mm,
                   agg_ref, xs_ref, dinv_ref, k1_ref, b1_ref, k2_ref,
                   out_ref):
    dvp = dinv_ref[...]
    a = agg_ref[...]
    x = xs_ref[...]
    u = jnp.concatenate(
        [(a[0, g] + a[1, g] + x[g]) * dvp for g in range(gin)], axis=1)
    hs = []
    for m in range(gmid):
        hm = jnp.dot(u, k1_ref[m], preferred_element_type=jnp.float32)
        hs.append(jnp.maximum(hm + b1_ref[m][None, :], 0.0))
    if two_mm:
        h = jnp.concatenate(hs, axis=1)
        for o in range(gout):
            out_ref[o] = dvp * jnp.dot(
                h, k2_ref[o], preferred_element_type=jnp.float32)
    else:
        for o in range(gout):
            out_ref[o] = dvp * hs[o]


def _tc3a_body(agg_ref, xs_ref, dinv_ref, b3_ref, ko_ref, bo_ref, z_ref):
    dvp = dinv_ref[...]
    a = agg_ref[...]
    x = xs_ref[...]
    h = jnp.concatenate(
        [jnp.maximum((a[0, g] + a[1, g] + x[g]) * dvp + b3_ref[g][None, :],
                     0.0)
         for g in range(4)], axis=1)
    z_ref[...] = jnp.dot(
        h, ko_ref[...], preferred_element_type=jnp.float32) + bo_ref[...][None, :]


def _tc3b_body(z_ref, batch_ref, out_ref, pool_ref):
    i = pl.program_id(0)
    z = z_ref[...]
    bb = batch_ref[...].reshape(BN)
    m = bb[None, :] == lax.broadcasted_iota(jnp.int32, (NGROUPS, BN), 0)
    neg = jnp.float32(-jnp.inf)

    @pl.when(i == 0)
    def _():
        pool_ref[...] = jnp.full((3, NGROUPS), neg, jnp.float32)

    for col in range(3):
        colmax = jnp.max(jnp.where(m, z[:, col][None, :], neg), axis=1)
        pool_ref[col, :] = jnp.maximum(pool_ref[col, :], colmax)

    @pl.when(i == NSTEP - 1)
    def _():
        p = pool_ref[...]
        mx = jnp.max(p, axis=0, keepdims=True)
        ex = jnp.exp(p - mx)
        out_ref[...] = ex / jnp.sum(ex, axis=0, keepdims=True)


_arb = pltpu.CompilerParams(dimension_semantics=("arbitrary",))


def _tc0(degv, xpk):
    return pl.pallas_call(
        _tc0_body,
        grid=(NSTEP,),
        in_specs=[
            pl.BlockSpec((2, RB, 128), lambda i: (0, i, 0)),
            pl.BlockSpec((3, RB, 128), lambda i: (0, i, 0)),
        ],
        out_specs=[
            pl.BlockSpec((RB, 128), lambda i: (i, 0)),
            pl.BlockSpec((3, RB, 128), lambda i: (0, i, 0)),
        ],
        out_shape=[
            jax.ShapeDtypeStruct((R, 128), jnp.float32),
            jax.ShapeDtypeStruct((3, R, 128), jnp.float32),
        ],
        compiler_params=_arb,
    )(degv, xpk)


def _tc_layer(aggv, xs, dinvp, k1, b1t, k2, gin, gmid, gout):
    two_mm = k2 is not None
    if k2 is None:
        k2 = jnp.zeros((1, 1, 1), jnp.float32)
    body = functools.partial(_tc_layer_body, gin, gmid, gout, two_mm)
    return pl.pallas_call(
        body,
        grid=(NSTEP,),
        in_specs=[
            pl.BlockSpec((2, gin, RB, 128), lambda i: (0, 0, i, 0)),
            pl.BlockSpec((gin, RB, 128), lambda i: (0, i, 0)),
            pl.BlockSpec((RB, 128), lambda i: (i, 0)),
            pl.BlockSpec(k1.shape, lambda i: (0, 0, 0)),
            pl.BlockSpec(b1t.shape, lambda i: (0, 0)),
            pl.BlockSpec(k2.shape, lambda i: (0, 0, 0)),
        ],
        out_specs=pl.BlockSpec((gout, RB, 128), lambda i: (0, i, 0)),
        out_shape=jax.ShapeDtypeStruct((gout, R, 128), jnp.float32),
        compiler_params=_arb,
    )(aggv, xs, dinvp, k1, b1t, k2)


def _tc3a(aggv, xs, dinvp, b3t, ko, bot):
    return pl.pallas_call(
        _tc3a_body,
        grid=(NSTEP,),
        in_specs=[
            pl.BlockSpec((2, 4, RB, 128), lambda i: (0, 0, i, 0)),
            pl.BlockSpec((4, RB, 128), lambda i: (0, i, 0)),
            pl.BlockSpec((RB, 128), lambda i: (i, 0)),
            pl.BlockSpec(b3t.shape, lambda i: (0, 0)),
            pl.BlockSpec(ko.shape, lambda i: (0, 0)),
            pl.BlockSpec(bot.shape, lambda i: (0,)),
        ],
        out_specs=pl.BlockSpec((RB, 24), lambda i: (i, 0)),
        out_shape=jax.ShapeDtypeStruct((R, 24), jnp.float32),
        compiler_params=_arb,
    )(aggv, xs, dinvp, b3t, ko, bot)


def _tc3b(z2, batchp):
    return pl.pallas_call(
        _tc3b_body,
        grid=(NSTEP,),
        in_specs=[
            pl.BlockSpec((BN, 3), lambda i: (i, 0)),
            pl.BlockSpec((1, 1, BN), lambda i: (i, 0, 0)),
        ],
        out_specs=pl.BlockSpec((3, NGROUPS), lambda i: (0, 0)),
        out_shape=jax.ShapeDtypeStruct((3, NGROUPS), jnp.float32),
        scratch_shapes=[pltpu.VMEM((3, NGROUPS), jnp.float32)],
        compiler_params=_arb,
    )(z2, batchp)


def _kron_stack(Wp, gin, gout):
    eye8 = jnp.eye(8, dtype=jnp.float32)
    cols = []
    for o in range(gout):
        rows = [jnp.kron(eye8, Wp[16 * g:16 * (g + 1), 16 * o:16 * (o + 1)])
                for g in range(gin)]
        cols.append(jnp.concatenate(rows, axis=0))
    return jnp.stack(cols, axis=0)


def _tile_bias(bp, g):
    return jnp.stack([jnp.tile(bp[16 * m:16 * (m + 1)], 8) for m in range(g)],
                     axis=0)


def kernel(x, edge_index, batch, W1, b1, W2, b2, W3, b3, Wo, bo):
    ed5 = edge_index.reshape(2, NW, NWCH, 1, CH)
    dst3 = edge_index[1].reshape(NW, NWIN, NCH, CH)

    xp48 = jnp.pad(x, ((0, NP - N), (0, 12)))
    xpk = jnp.stack(
        [xp48[:, 16 * g:16 * (g + 1)].reshape(R, 128) for g in range(3)],
        axis=0)
    batchp = jnp.pad(batch, (0, NP - N),
                     constant_values=NGROUPS).reshape(NSTEP, 1, BN)

    W1p = jnp.pad(W1, ((0, 12), (0, 5)))
    b1p = jnp.pad(b1, (0, 5))
    W2p = jnp.pad(W2, ((0, 5), (0, 10)))
    b2p = jnp.pad(b2, (0, 10))
    W3p = jnp.pad(W3, ((0, 10), (0, 14)))
    b3p = jnp.pad(b3, (0, 14))
    Wop = jnp.pad(Wo, ((0, 14), (0, 0)))

    K1 = _kron_stack(W1p, 3, 5)
    B1 = _tile_bias(b1p, 5)
    K2 = _kron_stack(W2p, 5, 10)
    B2 = _tile_bias(b2p, 10)
    K3 = _kron_stack(W3p, 10, 4)
    B3 = _tile_bias(b3p, 4)
    KO = jnp.concatenate(
        [jnp.kron(jnp.eye(8, dtype=jnp.float32), Wop[16 * g:16 * (g + 1), :])
         for g in range(4)], axis=0)
    BOT = jnp.tile(bo, 8)

    deg2 = _sc_degree(dst3)
    dinvp, xs1 = _tc0(deg2.reshape(2, R, 128), xpk)

    agg1 = _sc_agg3(xs1.reshape(3 * NP, 16), ed5)
    xs2 = _tc_layer(agg1.reshape(2, 3, R, 128), xs1, dinvp, K1, B1, None,
                    3, 5, 5)

    agg2 = _sc_agg5(xs2.reshape(5 * NP, 16), ed5)
    xs3 = _tc_layer(agg2.reshape(2, 5, R, 128), xs2, dinvp, K2, B2, K3,
                    5, 10, 4)

    agg3 = _sc_agg4(xs3.reshape(4 * NP, 16), ed5)
    zpk = _tc3a(agg3.reshape(2, 4, R, 128), xs3, dinvp, B3, KO, BOT)
    pooled_t = _tc3b(zpk.reshape(NP, 3), batchp)
    return pooled_t.T


# speedup vs baseline: 30.0911x; 1.4203x over previous
import functools

import jax
import jax.numpy as jnp
from jax import lax
from jax.experimental import pallas as pl
from jax.experimental import pallas as pl
from jax.experimental.pallas import tpu as pltpu
from jax.experimental.pallas import tpu_sc as plsc

N = 100000
NP = 100096
E = 1600000
NGROUPS = 64

NW = 32
EPT = E // NW
CH = 400
NWCH = EPT // CH
NWIN = 25
NCH = 5
SUB = 6272
SUBL = NP - 15 * SUB
ZR = 224

BN = 4352
NSTEP = NP // BN
RB = BN // 8
R = NP // 8

_mesh = plsc.VectorSubcoreMesh(core_axis_name="c", subcore_axis_name="s")
_sc_params = pltpu.CompilerParams(use_tc_tiling_on_sc=False)


def _make_sc_agg(G):
    @functools.partial(
        pl.kernel,
        out_type=jax.ShapeDtypeStruct((2, G, NP, 16), jnp.float32),
        mesh=_mesh,
        compiler_params=_sc_params,
        scratch_types=[
            pltpu.VMEM_SHARED((NP, 16), jnp.float32),
            pltpu.VMEM((3, 2, 1, CH), jnp.int32),
            pltpu.VMEM((3, CH), jnp.int32),
            pltpu.VMEM((3, CH), jnp.int32),
            pltpu.VMEM((3, CH, 16), jnp.float32),
            pltpu.VMEM((ZR, 16), jnp.float32),
            pltpu.SemaphoreType.DMA((3,)),
            pltpu.SemaphoreType.DMA((3,)),
            pltpu.SemaphoreType.DMA((3,)),
        ],
    )
    def agg(xs_hbm, ed_hbm, out_hbm,
            accum, ed_v, idx_v, didx_v, rows_v, z_v, esem, gsem, ssem):
        c = lax.axis_index("c")
        s = lax.axis_index("s")
        t = c * 16 + s

        def zb(i, carry):
            z_v[i] = jnp.zeros((16,), jnp.float32)
            return carry

        lax.fori_loop(0, ZR, zb, 0)

        def e_cp(k, b):
            return pltpu.make_async_copy(
                ed_hbm.at[:, t, k], ed_v.at[b], esem.at[b])

        def g_cp(b):
            return pltpu.make_async_copy(
                xs_hbm.at[idx_v.at[b]], rows_v.at[b], gsem.at[b])

        def s_cp(b):
            return pltpu.make_async_copy(
                rows_v.at[b], accum.at[didx_v.at[b]], ssem.at[b])

        for g in range(G):
            base = g * NP

            def idx_compute(b):
                for j in range(CH // 16):
                    sl = pl.ds(j * 16, 16)
                    idx_v[b, sl] = ed_v[b, 0, 0, sl] + base
                    didx_v[b, sl] = ed_v[b, 1, 0, sl]

            @pl.when(s < 15)
            def _():
                for k in range(SUB // ZR):
                    pltpu.sync_copy(
                        z_v, accum.at[pl.ds(s * SUB + k * ZR, ZR)])

            @pl.when(s == 15)
            def _():
                for k in range(SUBL // ZR):
                    pltpu.sync_copy(
                        z_v, accum.at[pl.ds(15 * SUB + k * ZR, ZR)])
                pltpu.sync_copy(
                    z_v.at[pl.ds(0, SUBL - ZR * (SUBL // ZR))],
                    accum.at[pl.ds(15 * SUB + ZR * (SUBL // ZR),
                                   SUBL - ZR * (SUBL // ZR))])

            plsc.subcore_barrier()

            e_cp(0, 0).start()
            e_cp(1, 1).start()
            e_cp(2, 2).start()
            e_cp(0, 0).wait()
            idx_compute(0)
            pltpu.async_copy(
                xs_hbm.at[idx_v.at[0]], rows_v.at[0], gsem.at[0])
            e_cp(1, 1).wait()
            idx_compute(1)
            pltpu.async_copy(
                xs_hbm.at[idx_v.at[1]], rows_v.at[1], gsem.at[1])

            def step(k, carry):
                b = lax.rem(k, 3)
                b2 = lax.rem(k + 2, 3)
                g_cp(b).wait()
                pltpu.async_copy(
                    rows_v.at[b], accum.at[didx_v.at[b]], ssem.at[b],
                    add=True)

                @pl.when(k + 2 < NWCH)
                def _():
                    @pl.when(k > 0)
                    def _():
                        s_cp(b2).wait()

                    e_cp(k + 2, b2).wait()
                    idx_compute(b2)
                    pltpu.async_copy(
                        xs_hbm.at[idx_v.at[b2]], rows_v.at[b2], gsem.at[b2])

                @pl.when(k + 3 < NWCH)
                def _():
                    e_cp(k + 3, b).start()

                return carry

            lax.fori_loop(0, NWCH, step, 0)
            s_cp(jnp.int32((NWCH - 3) % 3)).wait()
            s_cp(jnp.int32((NWCH - 2) % 3)).wait()
            s_cp(jnp.int32((NWCH - 1) % 3)).wait()
            plsc.subcore_barrier()

            @pl.when(s < 15)
            def _():
                pltpu.sync_copy(
                    accum.at[pl.ds(s * SUB, SUB)],
                    out_hbm.at[c, g, pl.ds(s * SUB, SUB)])

            @pl.when(s == 15)
            def _():
                pltpu.sync_copy(
                    accum.at[pl.ds(15 * SUB, SUBL)],
                    out_hbm.at[c, g, pl.ds(15 * SUB, SUBL)])

            plsc.subcore_barrier()

    return agg


@functools.partial(
    pl.kernel,
    out_type=jax.ShapeDtypeStruct((2, NP, 16), jnp.float32),
    mesh=_mesh,
    compiler_params=_sc_params,
    scratch_types=[
        pltpu.VMEM_SHARED((NP, 16), jnp.float32),
        pltpu.VMEM((1, CH), jnp.int32),
        pltpu.VMEM((CH, 16), jnp.float32),
        pltpu.VMEM((ZR, 16), jnp.float32),
    ],
)
def _sc_degree(ed_hbm, out_hbm, accum, dst_v, ones_v, z_v):
    c = lax.axis_index("c")
    s = lax.axis_index("s")
    t = c * 16 + s

    def zb(i, carry):
        z_v[i] = jnp.zeros((16,), jnp.float32)
        return carry

    lax.fori_loop(0, ZR, zb, 0)

    def ob(i, carry):
        ones_v[i] = jnp.ones((16,), jnp.float32)
        return carry

    lax.fori_loop(0, CH, ob, 0)

    @pl.when(s < 15)
    def _():
        for k in range(SUB // ZR):
            pltpu.sync_copy(z_v, accum.at[pl.ds(s * SUB + k * ZR, ZR)])

    @pl.when(s == 15)
    def _():
        for k in range(SUBL // ZR):
            pltpu.sync_copy(z_v, accum.at[pl.ds(15 * SUB + k * ZR, ZR)])
        pltpu.sync_copy(
            z_v.at[pl.ds(0, SUBL - ZR * (SUBL // ZR))],
            accum.at[pl.ds(15 * SUB + ZR * (SUBL // ZR),
                           SUBL - ZR * (SUBL // ZR))])

    plsc.subcore_barrier()

    def win(w, carry):
        pltpu.sync_copy(ed_hbm.at[1, t, w], dst_v)
        pltpu.sync_copy(ones_v, accum.at[dst_v.at[0]], add=True)
        return carry

    lax.fori_loop(0, NWCH, win, 0)
    plsc.subcore_barrier()

    @pl.when(s < 15)
    def _():
        pltpu.sync_copy(
            accum.at[pl.ds(s * SUB, SUB)],
            out_hbm.at[c, pl.ds(s * SUB, SUB)])

    @pl.when(s == 15)
    def _():
        pltpu.sync_copy(
            accum.at[pl.ds(15 * SUB, SUBL)],
            out_hbm.at[c, pl.ds(15 * SUB, SUBL)])


_sc_agg3 = _make_sc_agg(3)
_sc_agg5 = _make_sc_agg(5)
_sc_agg4 = _make_sc_agg(4)


def _tc0_body(deg_ref, xpk_ref, dinv_ref, xs1_ref):
    d = deg_ref[...]
    dvp = lax.rsqrt(d[0] + d[1] + 1.0)
    dinv_ref[...] = dvp
    for g in range(3):
        xs1_ref[g] = xpk_ref[g] * dvp


def _tc_layer_body(gin, gmid, gout, two_mm,
                   agg_ref, xs_ref, dinv_ref, k1_ref, b1_ref, k2_ref,
                   out_ref):
    dvp = dinv_ref[...]
    a = agg_ref[...]
    x = xs_ref[...]
    u = jnp.concatenate(
        [(a[0, g] + a[1, g] + x[g]) * dvp for g in range(gin)], axis=1)
    hs = []
    for m in range(gmid):
        hm = jnp.dot(u, k1_ref[m], preferred_element_type=jnp.float32)
        hs.append(jnp.maximum(hm + b1_ref[m][None, :], 0.0))
    if two_mm:
        h = jnp.concatenate(hs, axis=1)
        for o in range(gout):
            out_ref[o] = dvp * jnp.dot(
                h, k2_ref[o], preferred_element_type=jnp.float32)
    else:
        for o in range(gout):
            out_ref[o] = dvp * hs[o]


def _tc3a_body(agg_ref, xs_ref, dinv_ref, b3_ref, ko_ref, bo_ref, z_ref):
    dvp = dinv_ref[...]
    a = agg_ref[...]
    x = xs_ref[...]
    h = jnp.concatenate(
        [jnp.maximum((a[0, g] + a[1, g] + x[g]) * dvp + b3_ref[g][None, :],
                     0.0)
         for g in range(4)], axis=1)
    z_ref[...] = jnp.dot(
        h, ko_ref[...], preferred_element_type=jnp.float32) + bo_ref[...][None, :]


def _tc3b_body(z_ref, batch_ref, out_ref, pool_ref):
    i = pl.program_id(0)
    z = z_ref[...]
    bb = batch_ref[...].reshape(BN)
    m = bb[None, :] == lax.broadcasted_iota(jnp.int32, (NGROUPS, BN), 0)
    neg = jnp.float32(-jnp.inf)

    @pl.when(i == 0)
    def _():
        pool_ref[...] = jnp.full((3, NGROUPS), neg, jnp.float32)

    for col in range(3):
        colmax = jnp.max(jnp.where(m, z[:, col][None, :], neg), axis=1)
        pool_ref[col, :] = jnp.maximum(pool_ref[col, :], colmax)

    @pl.when(i == NSTEP - 1)
    def _():
        p = pool_ref[...]
        mx = jnp.max(p, axis=0, keepdims=True)
        ex = jnp.exp(p - mx)
        out_ref[...] = ex / jnp.sum(ex, axis=0, keepdims=True)


_arb = pltpu.CompilerParams(dimension_semantics=("arbitrary",))


def _tc0(degv, xpk):
    return pl.pallas_call(
        _tc0_body,
        grid=(NSTEP,),
        in_specs=[
            pl.BlockSpec((2, RB, 128), lambda i: (0, i, 0)),
            pl.BlockSpec((3, RB, 128), lambda i: (0, i, 0)),
        ],
        out_specs=[
            pl.BlockSpec((RB, 128), lambda i: (i, 0)),
            pl.BlockSpec((3, RB, 128), lambda i: (0, i, 0)),
        ],
        out_shape=[
            jax.ShapeDtypeStruct((R, 128), jnp.float32),
            jax.ShapeDtypeStruct((3, R, 128), jnp.float32),
        ],
        compiler_params=_arb,
    )(degv, xpk)


def _tc_layer(aggv, xs, dinvp, k1, b1t, k2, gin, gmid, gout):
    two_mm = k2 is not None
    if k2 is None:
        k2 = jnp.zeros((1, 1, 1), jnp.float32)
    body = functools.partial(_tc_layer_body, gin, gmid, gout, two_mm)
    return pl.pallas_call(
        body,
        grid=(NSTEP,),
        in_specs=[
            pl.BlockSpec((2, gin, RB, 128), lambda i: (0, 0, i, 0)),
            pl.BlockSpec((gin, RB, 128), lambda i: (0, i, 0)),
            pl.BlockSpec((RB, 128), lambda i: (i, 0)),
            pl.BlockSpec(k1.shape, lambda i: (0, 0, 0)),
            pl.BlockSpec(b1t.shape, lambda i: (0, 0)),
            pl.BlockSpec(k2.shape, lambda i: (0, 0, 0)),
        ],
        out_specs=pl.BlockSpec((gout, RB, 128), lambda i: (0, i, 0)),
        out_shape=jax.ShapeDtypeStruct((gout, R, 128), jnp.float32),
        compiler_params=_arb,
    )(aggv, xs, dinvp, k1, b1t, k2)


def _tc3a(aggv, xs, dinvp, b3t, ko, bot):
    return pl.pallas_call(
        _tc3a_body,
        grid=(NSTEP,),
        in_specs=[
            pl.BlockSpec((2, 4, RB, 128), lambda i: (0, 0, i, 0)),
            pl.BlockSpec((4, RB, 128), lambda i: (0, i, 0)),
            pl.BlockSpec((RB, 128), lambda i: (i, 0)),
            pl.BlockSpec(b3t.shape, lambda i: (0, 0)),
            pl.BlockSpec(ko.shape, lambda i: (0, 0)),
            pl.BlockSpec(bot.shape, lambda i: (0,)),
        ],
        out_specs=pl.BlockSpec((RB, 24), lambda i: (i, 0)),
        out_shape=jax.ShapeDtypeStruct((R, 24), jnp.float32),
        compiler_params=_arb,
    )(aggv, xs, dinvp, b3t, ko, bot)


def _tc3b(z2, batchp):
    return pl.pallas_call(
        _tc3b_body,
        grid=(NSTEP,),
        in_specs=[
            pl.BlockSpec((BN, 3), lambda i: (i, 0)),
            pl.BlockSpec((1, 1, BN), lambda i: (i, 0, 0)),
        ],
        out_specs=pl.BlockSpec((3, NGROUPS), lambda i: (0, 0)),
        out_shape=jax.ShapeDtypeStruct((3, NGROUPS), jnp.float32),
        scratch_shapes=[pltpu.VMEM((3, NGROUPS), jnp.float32)],
        compiler_params=_arb,
    )(z2, batchp)


def _kron_stack(Wp, gin, gout):
    eye8 = jnp.eye(8, dtype=jnp.float32)
    cols = []
    for o in range(gout):
        rows = [jnp.kron(eye8, Wp[16 * g:16 * (g + 1), 16 * o:16 * (o + 1)])
                for g in range(gin)]
        cols.append(jnp.concatenate(rows, axis=0))
    return jnp.stack(cols, axis=0)


def _tile_bias(bp, g):
    return jnp.stack([jnp.tile(bp[16 * m:16 * (m + 1)], 8) for m in range(g)],
                     axis=0)


def kernel(x, edge_index, batch, W1, b1, W2, b2, W3, b3, Wo, bo):
    ed5 = edge_index.reshape(2, NW, NWCH, 1, CH)

    xp48 = jnp.pad(x, ((0, NP - N), (0, 12)))
    xpk = jnp.stack(
        [xp48[:, 16 * g:16 * (g + 1)].reshape(R, 128) for g in range(3)],
        axis=0)
    batchp = jnp.pad(batch, (0, NP - N),
                     constant_values=NGROUPS).reshape(NSTEP, 1, BN)

    W1p = jnp.pad(W1, ((0, 12), (0, 5)))
    b1p = jnp.pad(b1, (0, 5))
    W2p = jnp.pad(W2, ((0, 5), (0, 10)))
    b2p = jnp.pad(b2, (0, 10))
    W3p = jnp.pad(W3, ((0, 10), (0, 14)))
    b3p = jnp.pad(b3, (0, 14))
    Wop = jnp.pad(Wo, ((0, 14), (0, 0)))

    K1 = _kron_stack(W1p, 3, 5)
    B1 = _tile_bias(b1p, 5)
    K2 = _kron_stack(W2p, 5, 10)
    B2 = _tile_bias(b2p, 10)
    K3 = _kron_stack(W3p, 10, 4)
    B3 = _tile_bias(b3p, 4)
    KO = jnp.concatenate(
        [jnp.kron(jnp.eye(8, dtype=jnp.float32), Wop[16 * g:16 * (g + 1), :])
         for g in range(4)], axis=0)
    BOT = jnp.tile(bo, 8)

    deg2 = _sc_degree(ed5)
    dinvp, xs1 = _tc0(deg2.reshape(2, R, 128), xpk)

    agg1 = _sc_agg3(xs1.reshape(3 * NP, 16), ed5)
    xs2 = _tc_layer(agg1.reshape(2, 3, R, 128), xs1, dinvp, K1, B1, None,
                    3, 5, 5)

    agg2 = _sc_agg5(xs2.reshape(5 * NP, 16), ed5)
    xs3 = _tc_layer(agg2.reshape(2, 5, R, 128), xs2, dinvp, K2, B2, K3,
                    5, 10, 4)

    agg3 = _sc_agg4(xs3.reshape(4 * NP, 16), ed5)
    zpk = _tc3a(agg3.reshape(2, 4, R, 128), xs3, dinvp, B3, KO, BOT)
    pooled_t = _tc3b(zpk.reshape(NP, 3), batchp)
    return pooled_t.T


# speedup vs baseline: 30.0924x; 1.0000x over previous
import functools

import jax
import jax.numpy as jnp
from jax import lax
from jax.experimental import pallas as pl
from jax.experimental.pallas import tpu as pltpu
from jax.experimental.pallas import tpu_sc as plsc

N = 100000
NP = 100096
E = 1600000
NGROUPS = 64

NW = 32
EPT = E // NW
CH = 400
NWCH = EPT // CH
NWIN = 25
NCH = 5
SUB = 6272
SUBL = NP - 15 * SUB
ZR = 224

BN = 4352
NSTEP = NP // BN
RB = BN // 8
R = NP // 8

_mesh = plsc.VectorSubcoreMesh(core_axis_name="c", subcore_axis_name="s")
_sc_params = pltpu.CompilerParams(use_tc_tiling_on_sc=False)


def _make_sc_agg(G):
    @functools.partial(
        pl.kernel,
        out_type=jax.ShapeDtypeStruct((2, G, NP, 16), jnp.float32),
        mesh=_mesh,
        compiler_params=_sc_params,
        scratch_types=[
            pltpu.VMEM_SHARED((NP, 16), jnp.float32),
            pltpu.VMEM((3, 2, 1, CH), jnp.int32),
            pltpu.VMEM((3, CH), jnp.int32),
            pltpu.VMEM((3, CH), jnp.int32),
            pltpu.VMEM((3, CH, 16), jnp.float32),
            pltpu.VMEM((ZR, 16), jnp.float32),
            pltpu.SemaphoreType.DMA((3,)),
            pltpu.SemaphoreType.DMA((3,)),
            pltpu.SemaphoreType.DMA((3,)),
        ],
    )
    def agg(xs_hbm, ed_hbm, out_hbm,
            accum, ed_v, idx_v, didx_v, rows_v, z_v, esem, gsem, ssem):
        c = lax.axis_index("c")
        s = lax.axis_index("s")
        t = c * 16 + s

        def zb(i, carry):
            z_v[i] = jnp.zeros((16,), jnp.float32)
            return carry

        lax.fori_loop(0, ZR, zb, 0)

        def e_cp(k, b):
            return pltpu.make_async_copy(
                ed_hbm.at[:, t, k], ed_v.at[b], esem.at[b])

        def g_cp(b):
            return pltpu.make_async_copy(
                xs_hbm.at[idx_v.at[b]], rows_v.at[b], gsem.at[b])

        def s_cp(b):
            return pltpu.make_async_copy(
                rows_v.at[b], accum.at[didx_v.at[b]], ssem.at[b])

        for g in range(G):
            base = g * NP

            def idx_compute(b):
                for j in range(CH // 16):
                    sl = pl.ds(j * 16, 16)
                    idx_v[b, sl] = ed_v[b, 0, 0, sl] + base
                    didx_v[b, sl] = ed_v[b, 1, 0, sl]

            @pl.when(s < 15)
            def _():
                for k in range(SUB // ZR):
                    pltpu.sync_copy(
                        z_v, accum.at[pl.ds(s * SUB + k * ZR, ZR)])

            @pl.when(s == 15)
            def _():
                for k in range(SUBL // ZR):
                    pltpu.sync_copy(
                        z_v, accum.at[pl.ds(15 * SUB + k * ZR, ZR)])
                pltpu.sync_copy(
                    z_v.at[pl.ds(0, SUBL - ZR * (SUBL // ZR))],
                    accum.at[pl.ds(15 * SUB + ZR * (SUBL // ZR),
                                   SUBL - ZR * (SUBL // ZR))])

            plsc.subcore_barrier()

            e_cp(0, 0).start()
            e_cp(1, 1).start()
            e_cp(2, 2).start()
            e_cp(0, 0).wait()
            idx_compute(0)
            pltpu.async_copy(
                xs_hbm.at[idx_v.at[0]], rows_v.at[0], gsem.at[0])
            e_cp(1, 1).wait()
            idx_compute(1)
            pltpu.async_copy(
                xs_hbm.at[idx_v.at[1]], rows_v.at[1], gsem.at[1])

            def step(k, carry):
                b = lax.rem(k, 3)
                b2 = lax.rem(k + 2, 3)
                g_cp(b).wait()
                pltpu.async_copy(
                    rows_v.at[b], accum.at[didx_v.at[b]], ssem.at[b],
                    add=True)

                @pl.when(k + 2 < NWCH)
                def _():
                    @pl.when(k > 0)
                    def _():
                        s_cp(b2).wait()

                    e_cp(k + 2, b2).wait()
                    idx_compute(b2)
                    pltpu.async_copy(
                        xs_hbm.at[idx_v.at[b2]], rows_v.at[b2], gsem.at[b2])

                @pl.when(k + 3 < NWCH)
                def _():
                    e_cp(k + 3, b).start()

                return carry

            lax.fori_loop(0, NWCH, step, 0)
            s_cp(jnp.int32((NWCH - 3) % 3)).wait()
            s_cp(jnp.int32((NWCH - 2) % 3)).wait()
            s_cp(jnp.int32((NWCH - 1) % 3)).wait()
            plsc.subcore_barrier()

            @pl.when(s < 15)
            def _():
                pltpu.sync_copy(
                    accum.at[pl.ds(s * SUB, SUB)],
                    out_hbm.at[c, g, pl.ds(s * SUB, SUB)])

            @pl.when(s == 15)
            def _():
                pltpu.sync_copy(
                    accum.at[pl.ds(15 * SUB, SUBL)],
                    out_hbm.at[c, g, pl.ds(15 * SUB, SUBL)])

            plsc.subcore_barrier()

    return agg


@functools.partial(
    pl.kernel,
    out_type=jax.ShapeDtypeStruct((2, NP, 16), jnp.float32),
    mesh=_mesh,
    compiler_params=_sc_params,
    scratch_types=[
        pltpu.VMEM_SHARED((NP, 16), jnp.float32),
        pltpu.VMEM((1, CH), jnp.int32),
        pltpu.VMEM((CH, 16), jnp.float32),
        pltpu.VMEM((ZR, 16), jnp.float32),
    ],
)
def _sc_degree(ed_hbm, out_hbm, accum, dst_v, ones_v, z_v):
    c = lax.axis_index("c")
    s = lax.axis_index("s")
    t = c * 16 + s

    def zb(i, carry):
        z_v[i] = jnp.zeros((16,), jnp.float32)
        return carry

    lax.fori_loop(0, ZR, zb, 0)

    def ob(i, carry):
        ones_v[i] = jnp.ones((16,), jnp.float32)
        return carry

    lax.fori_loop(0, CH, ob, 0)

    @pl.when(s < 15)
    def _():
        for k in range(SUB // ZR):
            pltpu.sync_copy(z_v, accum.at[pl.ds(s * SUB + k * ZR, ZR)])

    @pl.when(s == 15)
    def _():
        for k in range(SUBL // ZR):
            pltpu.sync_copy(z_v, accum.at[pl.ds(15 * SUB + k * ZR, ZR)])
        pltpu.sync_copy(
            z_v.at[pl.ds(0, SUBL - ZR * (SUBL // ZR))],
            accum.at[pl.ds(15 * SUB + ZR * (SUBL // ZR),
                           SUBL - ZR * (SUBL // ZR))])

    plsc.subcore_barrier()

    def win(w, carry):
        pltpu.sync_copy(ed_hbm.at[1, t, w], dst_v)
        pltpu.sync_copy(ones_v, accum.at[dst_v.at[0]], add=True)
        return carry

    lax.fori_loop(0, NWCH, win, 0)
    plsc.subcore_barrier()

    @pl.when(s < 15)
    def _():
        pltpu.sync_copy(
            accum.at[pl.ds(s * SUB, SUB)],
            out_hbm.at[c, pl.ds(s * SUB, SUB)])

    @pl.when(s == 15)
    def _():
        pltpu.sync_copy(
            accum.at[pl.ds(15 * SUB, SUBL)],
            out_hbm.at[c, pl.ds(15 * SUB, SUBL)])


_sc_agg3 = _make_sc_agg(3)
_sc_agg5 = _make_sc_agg(5)
_sc_agg4 = _make_sc_agg(4)


def _tc0_body(deg_ref, xpk_ref, dinv_ref, xs1_ref):
    d = deg_ref[...]
    dvp = lax.rsqrt(d[0] + d[1] + 1.0)
    dinv_ref[...] = dvp
    for g in range(3):
        xs1_ref[g] = xpk_ref[g] * dvp


def _tc_layer_body(gin, gmid, gout, two_mm,
                   agg_ref, xs_ref, dinv_ref, k1_ref, b1_ref, k2_ref,
                   out_ref):
    dvp = dinv_ref[...]
    a = agg_ref[...]
    x = xs_ref[...]
    u = jnp.concatenate(
        [(a[0, g] + a[1, g] + x[g]) * dvp for g in range(gin)], axis=1)
    hs = []
    for m in range(gmid):
        hm = jnp.dot(u, k1_ref[m], preferred_element_type=jnp.float32)
        hs.append(jnp.maximum(hm + b1_ref[m][None, :], 0.0))
    if two_mm:
        h = jnp.concatenate(hs, axis=1)
        for o in range(gout):
            out_ref[o] = dvp * jnp.dot(
                h, k2_ref[o], preferred_element_type=jnp.float32)
    else:
        for o in range(gout):
            out_ref[o] = dvp * hs[o]


def _tc3a_body(agg_ref, xs_ref, dinv_ref, b3_ref, ko_ref, bo_ref, z_ref):
    dvp = dinv_ref[...]
    a = agg_ref[...]
    x = xs_ref[...]
    h = jnp.concatenate(
        [jnp.maximum((a[0, g] + a[1, g] + x[g]) * dvp + b3_ref[g][None, :],
                     0.0)
         for g in range(4)], axis=1)
    z_ref[...] = jnp.dot(
        h, ko_ref[...], preferred_element_type=jnp.float32) + bo_ref[...][None, :]


def _tc3b_body(z_ref, batch_ref, out_ref, pool_ref):
    i = pl.program_id(0)
    z = z_ref[...]
    bb = batch_ref[...].reshape(BN)
    m = bb[None, :] == lax.broadcasted_iota(jnp.int32, (NGROUPS, BN), 0)
    neg = jnp.float32(-jnp.inf)

    @pl.when(i == 0)
    def _():
        pool_ref[...] = jnp.full((3, NGROUPS), neg, jnp.float32)

    for col in range(3):
        colmax = jnp.max(jnp.where(m, z[:, col][None, :], neg), axis=1)
        pool_ref[col, :] = jnp.maximum(pool_ref[col, :], colmax)

    @pl.when(i == NSTEP - 1)
    def _():
        p = pool_ref[...]
        mx = jnp.max(p, axis=0, keepdims=True)
        ex = jnp.exp(p - mx)
        out_ref[...] = ex / jnp.sum(ex, axis=0, keepdims=True)


_arb = pltpu.CompilerParams(dimension_semantics=("arbitrary",))


def _tc0(degv, xpk):
    return pl.pallas_call(
        _tc0_body,
        grid=(NSTEP,),
        in_specs=[
            pl.BlockSpec((2, RB, 128), lambda i: (0, i, 0)),
            pl.BlockSpec((3, RB, 128), lambda i: (0, i, 0)),
        ],
        out_specs=[
            pl.BlockSpec((RB, 128), lambda i: (i, 0)),
            pl.BlockSpec((3, RB, 128), lambda i: (0, i, 0)),
        ],
        out_shape=[
            jax.ShapeDtypeStruct((R, 128), jnp.float32),
            jax.ShapeDtypeStruct((3, R, 128), jnp.float32),
        ],
        compiler_params=_arb,
    )(degv, xpk)


def _tc_layer(aggv, xs, dinvp, k1, b1t, k2, gin, gmid, gout):
    two_mm = k2 is not None
    if k2 is None:
        k2 = jnp.zeros((1, 1, 1), jnp.float32)
    body = functools.partial(_tc_layer_body, gin, gmid, gout, two_mm)
    return pl.pallas_call(
        body,
        grid=(NSTEP,),
        in_specs=[
            pl.BlockSpec((2, gin, RB, 128), lambda i: (0, 0, i, 0)),
            pl.BlockSpec((gin, RB, 128), lambda i: (0, i, 0)),
            pl.BlockSpec((RB, 128), lambda i: (i, 0)),
            pl.BlockSpec(k1.shape, lambda i: (0, 0, 0)),
            pl.BlockSpec(b1t.shape, lambda i: (0, 0)),
            pl.BlockSpec(k2.shape, lambda i: (0, 0, 0)),
        ],
        out_specs=pl.BlockSpec((gout, RB, 128), lambda i: (0, i, 0)),
        out_shape=jax.ShapeDtypeStruct((gout, R, 128), jnp.float32),
        compiler_params=_arb,
    )(aggv, xs, dinvp, k1, b1t, k2)


def _tc3a(aggv, xs, dinvp, b3t, ko, bot):
    return pl.pallas_call(
        _tc3a_body,
        grid=(NSTEP,),
        in_specs=[
            pl.BlockSpec((2, 4, RB, 128), lambda i: (0, 0, i, 0)),
            pl.BlockSpec((4, RB, 128), lambda i: (0, i, 0)),
            pl.BlockSpec((RB, 128), lambda i: (i, 0)),
            pl.BlockSpec(b3t.shape, lambda i: (0, 0)),
            pl.BlockSpec(ko.shape, lambda i: (0, 0)),
            pl.BlockSpec(bot.shape, lambda i: (0,)),
        ],
        out_specs=pl.BlockSpec((RB, 24), lambda i: (i, 0)),
        out_shape=jax.ShapeDtypeStruct((R, 24), jnp.float32),
        compiler_params=_arb,
    )(aggv, xs, dinvp, b3t, ko, bot)


def _tc3b(z2, batchp):
    return pl.pallas_call(
        _tc3b_body,
        grid=(NSTEP,),
        in_specs=[
            pl.BlockSpec((BN, 3), lambda i: (i, 0)),
            pl.BlockSpec((1, 1, BN), lambda i: (i, 0, 0)),
        ],
        out_specs=pl.BlockSpec((3, NGROUPS), lambda i: (0, 0)),
        out_shape=jax.ShapeDtypeStruct((3, NGROUPS), jnp.float32),
        scratch_shapes=[pltpu.VMEM((3, NGROUPS), jnp.float32)],
        compiler_params=_arb,
    )(z2, batchp)


def _kron_stack(Wp, gin, gout):
    eye8 = jnp.eye(8, dtype=jnp.float32)
    cols = []
    for o in range(gout):
        rows = [jnp.kron(eye8, Wp[16 * g:16 * (g + 1), 16 * o:16 * (o + 1)])
                for g in range(gin)]
        cols.append(jnp.concatenate(rows, axis=0))
    return jnp.stack(cols, axis=0)


def _tile_bias(bp, g):
    return jnp.stack([jnp.tile(bp[16 * m:16 * (m + 1)], 8) for m in range(g)],
                     axis=0)


def kernel(x, edge_index, batch, W1, b1, W2, b2, W3, b3, Wo, bo):
    ed5 = edge_index.reshape(2, NW, NWCH, 1, CH)

    xp48 = jnp.pad(x, ((0, NP - N), (0, 12)))
    xpk = jnp.stack(
        [xp48[:, 16 * g:16 * (g + 1)].reshape(R, 128) for g in range(3)],
        axis=0)
    batchp = jnp.pad(batch, (0, NP - N),
                     constant_values=NGROUPS).reshape(NSTEP, 1, BN)

    W1p = jnp.pad(W1, ((0, 12), (0, 5)))
    b1p = jnp.pad(b1, (0, 5))
    W2p = jnp.pad(W2, ((0, 5), (0, 10)))
    b2p = jnp.pad(b2, (0, 10))
    W3p = jnp.pad(W3, ((0, 10), (0, 14)))
    b3p = jnp.pad(b3, (0, 14))
    Wop = jnp.pad(Wo, ((0, 14), (0, 0)))

    K1 = _kron_stack(W1p, 3, 5)
    B1 = _tile_bias(b1p, 5)
    K2 = _kron_stack(W2p, 5, 10)
    B2 = _tile_bias(b2p, 10)
    K3 = _kron_stack(W3p, 10, 4)
    B3 = _tile_bias(b3p, 4)
    KO = jnp.concatenate(
        [jnp.kron(jnp.eye(8, dtype=jnp.float32), Wop[16 * g:16 * (g + 1), :])
         for g in range(4)], axis=0)
    BOT = jnp.tile(bo, 8)

    deg2 = _sc_degree(ed5)
    dinvp, xs1 = _tc0(deg2.reshape(2, R, 128), xpk)

    agg1 = _sc_agg3(xs1.reshape(3 * NP, 16), ed5)
    xs2 = _tc_layer(agg1.reshape(2, 3, R, 128), xs1, dinvp, K1, B1, None,
                    3, 5, 5)

    agg2 = _sc_agg5(xs2.reshape(5 * NP, 16), ed5)
    xs3 = _tc_layer(agg2.reshape(2, 5, R, 128), xs2, dinvp, K2, B2, K3,
                    5, 10, 4)

    agg3 = _sc_agg4(xs3.reshape(4 * NP, 16), ed5)
    zpk = _tc3a(agg3.reshape(2, 4, R, 128), xs3, dinvp, B3, KO, BOT)
    pooled_t = _tc3b(zpk.reshape(NP, 3), batchp)
    return pooled_t.T
